# Initial kernel scaffold; baseline (speedup 1.0000x reference)
import jax
import jax.numpy as jnp
from jax.experimental import pallas as pl


def kernel(x, edge_index, edge_type, edge_weight, W0, asrc0, adst0, ete0, g0, b0, W1, asrc1, adst1, ete1, g1, b1, W2, asrc2, adst2, ete2, g2, b2, sW1, sb1, sW2, sb2, rW1_0, rb1_0, rW2_0, rb2_0, rW1_1, rb1_1, rW2_1, rb2_1, baseline):
    raise NotImplementedError("write your pallas kernel here")


# speedup vs baseline: 6.0271x; 6.0271x over previous
import functools

import jax
import jax.numpy as jnp
import numpy as np
from jax import lax
from jax.experimental import pallas as pl
from jax.experimental.pallas import tpu as pltpu
from jax.experimental.pallas import tpu_sc as plsc

F32 = jnp.float32
I32 = jnp.int32

NC = 2
NS = 16
NW = NC * NS

GRP = 64
GPC = 4
CH = GRP * GPC

_HIGH = lax.Precision.HIGHEST
_Z = np.int32(0)


def _elu(x):
    return jnp.where(x > 0, x, (jnp.exp(x) - 1.0))


def _selu(x):
    scale = 1.0507009873554805
    alpha = 1.6732632423543772
    return scale * jnp.where(x > 0, x, alpha * (jnp.exp(x) - 1.0))


def _softplus(x):
    return jnp.maximum(x, 0.0) + jnp.log(1.0 + jnp.exp(-jnp.abs(x)))


def _tc_pre(x, WT, A, bn):
    n = x.shape[0]

    def body(x_ref, wt_ref, a_ref, h_ref, sd_ref):
        hb = jnp.dot(x_ref[...], wt_ref[...], preferred_element_type=F32,
                     precision=_HIGH)
        h_ref[...] = hb
        sd_ref[...] = jnp.dot(hb, a_ref[...], preferred_element_type=F32,
                              precision=_HIGH)

    grid = (n // bn,)
    return pl.pallas_call(
        body,
        grid=grid,
        in_specs=[
            pl.BlockSpec((bn, 128), lambda i: (i, _Z)),
            pl.BlockSpec((128, 128), lambda i: (_Z, _Z)),
            pl.BlockSpec((128, 16), lambda i: (_Z, _Z)),
        ],
        out_specs=[
            pl.BlockSpec((bn, 128), lambda i: (i, _Z)),
            pl.BlockSpec((bn, 16), lambda i: (i, _Z)),
        ],
        out_shape=[
            jax.ShapeDtypeStruct((n, 128), F32),
            jax.ShapeDtypeStruct((n, 16), F32),
        ],
    )(x, WT, A)


def _tc_mid(praw, psum, res, g, b, R, WT, A, bn):
    n = res.shape[0]

    def body(pr_ref, ps_ref, res_ref, g_ref, b_ref, r_ref, wt_ref, a_ref,
             act_ref, h_ref, sd_ref):
        pr = pr_ref[0] + pr_ref[1]
        sm = jnp.sum(ps_ref[...], axis=0)
        smr = jnp.dot(sm, r_ref[...], preferred_element_type=F32,
                      precision=_HIGH)
        out = pr / (smr + 1e-10)
        out = out + res_ref[...]
        mu = jnp.mean(out, axis=-1, keepdims=True)
        var = jnp.mean((out - mu) ** 2, axis=-1, keepdims=True)
        out = (out - mu) / jnp.sqrt(var + 1e-5) * g_ref[...] + b_ref[...]
        act = _elu(out)
        act_ref[...] = act
        hb = jnp.dot(act, wt_ref[...], preferred_element_type=F32,
                     precision=_HIGH)
        h_ref[...] = hb
        sd_ref[...] = jnp.dot(hb, a_ref[...], preferred_element_type=F32,
                              precision=_HIGH)

    grid = (n // bn,)
    return pl.pallas_call(
        body,
        grid=grid,
        in_specs=[
            pl.BlockSpec((2, bn, 128), lambda i: (_Z, i, _Z)),
            pl.BlockSpec((NW, bn, 8), lambda i: (_Z, i, _Z)),
            pl.BlockSpec((bn, 128), lambda i: (i, _Z)),
            pl.BlockSpec((1, 128), lambda i: (_Z, _Z)),
            pl.BlockSpec((1, 128), lambda i: (_Z, _Z)),
            pl.BlockSpec((8, 128), lambda i: (_Z, _Z)),
            pl.BlockSpec((128, 128), lambda i: (_Z, _Z)),
            pl.BlockSpec((128, 16), lambda i: (_Z, _Z)),
        ],
        out_specs=[
            pl.BlockSpec((bn, 128), lambda i: (i, _Z)),
            pl.BlockSpec((bn, 128), lambda i: (i, _Z)),
            pl.BlockSpec((bn, 16), lambda i: (i, _Z)),
        ],
        out_shape=[
            jax.ShapeDtypeStruct((n, 128), F32),
            jax.ShapeDtypeStruct((n, 128), F32),
            jax.ShapeDtypeStruct((n, 16), F32),
        ],
    )(praw, psum, res, g, b, R, WT, A)


def _tc_head(praw, psum, res, g, b, R, sW1T, sb1, sW2T, sb2,
             rW1_0T, rb1_0, rW2_0T, rb2_0, rW1_1T, rb1_1, rW2_1T, rb2_1,
             base0, base1, U, Z, e0row, bn):
    n = res.shape[0]
    T = U.shape[0]

    def body(pr_ref, ps_ref, res_ref, g_ref, b_ref, r_ref,
             sw1_ref, sb1_ref, sw2_ref, sb2_ref,
             rw10_ref, rb10_ref, rw20_ref, rb20_ref,
             rw11_ref, rb11_ref, rw21_ref, rb21_ref,
             b0_ref, b1_ref, u_ref, z_ref, e0_ref,
             hz0_ref, hz1_ref, surv_ref, cif0_ref, cif1_ref):
        pr = pr_ref[0] + pr_ref[1]
        sm = jnp.sum(ps_ref[...], axis=0)
        smr = jnp.dot(sm, r_ref[...], preferred_element_type=F32,
                      precision=_HIGH)
        out = pr / (smr + 1e-10)
        out = out + res_ref[...]
        mu = jnp.mean(out, axis=-1, keepdims=True)
        var = jnp.mean((out - mu) ** 2, axis=-1, keepdims=True)
        out = (out - mu) / jnp.sqrt(var + 1e-5) * g_ref[...] + b_ref[...]

        s = _selu(jnp.dot(out, sw1_ref[...], preferred_element_type=F32,
                          precision=_HIGH) + sb1_ref[...])
        s = _selu(jnp.dot(s, sw2_ref[...], preferred_element_type=F32,
                          precision=_HIGH) + sb2_ref[...])
        q0 = _selu(jnp.dot(s, rw10_ref[...], preferred_element_type=F32,
                           precision=_HIGH) + rb10_ref[...])
        lh0 = jnp.dot(q0, rw20_ref[...], preferred_element_type=F32,
                      precision=_HIGH) + rb20_ref[...] + b0_ref[...]
        q1 = _selu(jnp.dot(s, rw11_ref[...], preferred_element_type=F32,
                           precision=_HIGH) + rb11_ref[...])
        lh1 = jnp.dot(q1, rw21_ref[...], preferred_element_type=F32,
                      precision=_HIGH) + rb21_ref[...] + b1_ref[...]
        hz0 = _softplus(lh0)
        hz1 = _softplus(lh1)
        hz0_ref[...] = hz0
        hz1_ref[...] = hz1
        total = hz0 + hz1
        cum = jnp.dot(total, u_ref[...], preferred_element_type=F32,
                      precision=_HIGH)
        surv = jnp.exp(-cum)
        surv_ref[...] = surv
        shifted = jnp.dot(surv, z_ref[...], preferred_element_type=F32,
                          precision=_HIGH) + e0_ref[...]
        cif0_ref[...] = jnp.dot(hz0 * shifted, u_ref[...],
                                preferred_element_type=F32, precision=_HIGH)
        cif1_ref[...] = jnp.dot(hz1 * shifted, u_ref[...],
                                preferred_element_type=F32, precision=_HIGH)

    grid = (n // bn,)
    full = lambda r, c: pl.BlockSpec((r, c), lambda i: (_Z, _Z))
    return pl.pallas_call(
        body,
        grid=grid,
        in_specs=[
            pl.BlockSpec((2, bn, 128), lambda i: (_Z, i, _Z)),
            pl.BlockSpec((NW, bn, 8), lambda i: (_Z, i, _Z)),
            pl.BlockSpec((bn, 128), lambda i: (i, _Z)),
            full(1, 128), full(1, 128), full(8, 128),
            full(128, 128), full(1, 128), full(128, 128), full(1, 128),
            full(128, 64), full(1, 64), full(64, T), full(1, T),
            full(128, 64), full(1, 64), full(64, T), full(1, T),
            full(1, T), full(1, T), full(T, T), full(T, T), full(1, T),
        ],
        out_specs=[pl.BlockSpec((bn, T), lambda i: (i, _Z))] * 5,
        out_shape=[jax.ShapeDtypeStruct((n, T), F32)] * 5,
    )(praw, psum, res, g, b, R, sW1T, sb1, sW2T, sb2,
      rW1_0T, rb1_0, rW2_0T, rb2_0, rW1_1T, rb1_1, rW2_1T, rb2_1,
      base0, base1, U, Z, e0row)


def _sc_aggregate(h, sd_pad, src2d, dst2d, et2d, ew2d, ete1, zeros_pad,
                  zeros8, H, base, half):
    e_rows = src2d.shape[0]
    rows_per_tile = half // NS
    chunks_per_worker = e_rows // (NW * GPC)

    mesh = plsc.VectorSubcoreMesh(core_axis_name="c", subcore_axis_name="s")

    @functools.partial(
        pl.kernel,
        mesh=mesh,
        out_type=(
            jax.ShapeDtypeStruct((NC, half, 128), F32),
            jax.ShapeDtypeStruct((NC, NS, half * 8), F32),
        ),
        scratch_types=[
            pltpu.VMEM((GPC, GRP), I32),
            pltpu.VMEM((GPC, GRP), I32),
            pltpu.VMEM((GPC, GRP), I32),
            pltpu.VMEM((GPC, GRP), I32),
            pltpu.VMEM((GPC, GRP), F32),
            pltpu.VMEM((GRP * 8,), F32),
            pltpu.VMEM((GRP, 128), F32),
            pltpu.VMEM((GRP, 16), F32),
            pltpu.VMEM((8, 128), F32),
            pltpu.VMEM((half * 8,), F32),
            pltpu.VMEM_SHARED((half, 128), F32),
            pltpu.SemaphoreType.DMA,
        ],
        compiler_params=pltpu.CompilerParams(needs_layout_passes=False),
    )
    def agg(h_hbm, sd_hbm, src_hbm, dst_hbm, et_hbm, ew_hbm, ete_hbm, z_hbm,
            z8_hbm, praw_hbm, psum_hbm,
            srcv, dstv, dstl, etv, ewv, ssb, hg, wg, etev, psum_t, praw_s,
            sem):
        cid = lax.axis_index("c")
        sid = lax.axis_index("s")
        wid = sid * NC + cid
        r0 = sid * rows_per_tile

        pltpu.sync_copy(z_hbm.at[pl.ds(0, GRP)], hg)
        pltpu.sync_copy(z8_hbm.at[pl.ds(0, GRP)], wg)
        offs = 0
        nfull, rem = divmod(rows_per_tile, GRP)
        for sz in [GRP] * nfull + ([rem] if rem else []):
            pltpu.sync_copy(hg.at[pl.ds(0, sz)],
                            praw_s.at[pl.ds(r0 + offs, sz)])
            offs += sz
        zvec = jnp.zeros((16,), F32)

        @pl.loop(jnp.int32(0), jnp.int32(half // 2))
        def _z(i):
            psum_t[pl.ds(i * 16, 16)] = zvec
        pltpu.sync_copy(ete_hbm, etev)
        plsc.subcore_barrier()

        lanes = lax.iota(I32, 16)

        @pl.loop(jnp.int32(0), jnp.int32(chunks_per_worker))
        def _chunk(c):
            row0 = (wid * chunks_per_worker + c) * GPC
            pltpu.sync_copy(src_hbm.at[pl.ds(row0, GPC)], srcv)
            pltpu.sync_copy(dst_hbm.at[pl.ds(row0, GPC)], dstv)
            pltpu.sync_copy(et_hbm.at[pl.ds(row0, GPC)], etv)
            pltpu.sync_copy(ew_hbm.at[pl.ds(row0, GPC)], ewv)

            @pl.loop(jnp.int32(0), jnp.int32(GPC))
            def _group(g):
                @pl.loop(jnp.int32(0), jnp.int32(GRP // 16))
                def _l(q):
                    d16 = dstv[g, pl.ds(q * 16, 16)]
                    loc = jnp.clip(d16 - base, 0, half - 1)
                    dstl[g, pl.ds(q * 16, 16)] = loc

                pltpu.async_copy(sd_hbm.at[srcv.at[g]], hg, sem).wait()

                @pl.loop(jnp.int32(0), jnp.int32(GRP))
                def _x(e):
                    v = hg[e, pl.ds(0, 16)]
                    plsc.store_scatter(ssb, [e * 8 + (lanes & 7)], v,
                                       mask=lanes < 8)

                pltpu.async_copy(sd_hbm.at[dstv.at[g]], hg, sem).wait()

                for hh in range(H):
                    @pl.loop(jnp.int32(0), jnp.int32(GRP // 16))
                    def _q(q):
                        e0 = q * 16
                        ev = lanes + e0
                        hv = jnp.full((16,), hh, I32)
                        ss = plsc.load_gather(ssb, [ev * 8 + hh])
                        dd = plsc.load_gather(hg, [ev, hv + 8])
                        lg = ss + dd
                        lg = jnp.where(lg > 0, lg, 0.2 * lg)
                        etq = etv[g, pl.ds(e0, 16)]
                        ee = plsc.load_gather(etev, [etq, hv])
                        w16 = jnp.exp(lg * ee * ewv[g, pl.ds(e0, 16)])
                        d16 = dstv[g, pl.ds(e0, 16)]
                        ok = (d16 >= base) & (d16 < base + half)
                        w16 = jnp.where(ok, w16, 0.0)
                        plsc.store_scatter(wg, [ev, hv], w16)

                pltpu.async_copy(h_hbm.at[srcv.at[g]], hg, sem).wait()

                @pl.loop(jnp.int32(0), jnp.int32(GRP))
                def _e(e):
                    ef = jnp.full((16,), e, I32)
                    for k in range(8):
                        kf = jnp.full((16,), k if H == 8 else 0, I32)
                        s = plsc.load_gather(wg, [ef, kf])
                        hg[e, pl.ds(k * 16, 16)] = hg[e, pl.ds(k * 16, 16)] * s

                @pl.loop(jnp.int32(0), jnp.int32(GRP))
                def _w(e):
                    loc = plsc.load_gather(
                        dstl, [jnp.full((16,), g, I32), jnp.full((16,), e, I32)])
                    plsc.addupdate_scatter(psum_t, [loc * 8 + (lanes & 7)],
                                           wg[e, pl.ds(0, 16)],
                                           mask=lanes < 8)

                pltpu.sync_copy(hg, praw_s.at[dstl.at[g]], add=True)

        plsc.subcore_barrier()
        offs2 = 0
        for sz in [GRP] * nfull + ([rem] if rem else []):
            pltpu.sync_copy(praw_s.at[pl.ds(r0 + offs2, sz)],
                            praw_hbm.at[cid, pl.ds(r0 + offs2, sz)])
            offs2 += sz
        pltpu.sync_copy(psum_t, psum_hbm.at[cid, sid])

    return agg(h, sd_pad, src2d, dst2d, et2d, ew2d, ete1, zeros_pad, zeros8)


def _block_diag_cols(a):
    Hh, D = a.shape
    out = jnp.zeros((128, 8), F32)
    for hh in range(Hh):
        out = out.at[hh * D:(hh + 1) * D, hh].set(a[hh])
    return out


def kernel(x, edge_index, edge_type, edge_weight, W0, asrc0, adst0, ete0, g0, b0, W1, asrc1, adst1, ete1, g1, b1, W2, asrc2, adst2, ete2, g2, b2, sW1, sb1, sW2, sb2, rW1_0, rb1_0, rW2_0, rb2_0, rW1_1, rb1_1, rW2_1, rb2_1, baseline):
    n, feat = x.shape
    e = edge_index.shape[1]
    T = baseline.shape[1]

    x = x.astype(F32)
    bn = 1000

    e_pad = ((e + NW * CH - 1) // (NW * CH)) * (NW * CH)
    half = -(-(n + 1) // 2 // 128) * 128
    n_pad = 2 * half
    npad_extra = e_pad - e

    src = edge_index[0].astype(I32)
    dst = edge_index[1].astype(I32)
    et = edge_type.astype(I32)
    ew = edge_weight.astype(F32)
    src_p = jnp.concatenate([src, jnp.zeros((npad_extra,), I32)])
    dst_p = jnp.concatenate([dst, jnp.full((npad_extra,), n, I32)])
    et_p = jnp.concatenate([et, jnp.zeros((npad_extra,), I32)])
    ew_p = jnp.concatenate([ew, jnp.zeros((npad_extra,), F32)])
    src2d = src_p.reshape(e_pad // GRP, GRP)
    dst2d = dst_p.reshape(e_pad // GRP, GRP)
    et2d = et_p.reshape(e_pad // GRP, GRP)
    ew2d = ew_p.reshape(e_pad // GRP, GRP)

    zeros_pad = jnp.zeros((128, 128), F32)
    zeros8 = jnp.zeros((128, 16), F32)

    A0 = jnp.concatenate([_block_diag_cols(asrc0), _block_diag_cols(adst0)], 1)
    A1 = jnp.concatenate([_block_diag_cols(asrc1), _block_diag_cols(adst1)], 1)
    A2 = jnp.concatenate([_block_diag_cols(asrc2), _block_diag_cols(adst2)], 1)

    R8 = jnp.zeros((8, 128), F32)
    for hh in range(8):
        R8 = R8.at[hh, hh * 16:(hh + 1) * 16].set(1.0)
    R1 = jnp.zeros((8, 128), F32).at[0, :].set(1.0)

    ete0_1 = jnp.pad((1.0 + ete0).astype(F32), ((0, 0), (0, 120)))
    ete2_1 = jnp.zeros((8, 128), F32).at[:, 0].set(1.0 + ete2[:, 0])

    tt = jnp.arange(T)
    U = (tt[:, None] <= tt[None, :]).astype(F32)
    Z = (tt[None, :] == tt[:, None] + 1).astype(F32)
    e0row = (tt[None, :] == 0).astype(F32)

    g0r, b0r = g0.reshape(1, 128), b0.reshape(1, 128)
    g1r, b1r = g1.reshape(1, 128), b1.reshape(1, 128)
    g2r, b2r = g2.reshape(1, 128), b2.reshape(1, 128)

    pad_sd = lambda sd: jnp.pad(sd, ((0, n_pad - n), (0, 112)))

    ete1_1 = jnp.pad((1.0 + ete1).astype(F32), ((0, 0), (0, 120)))

    def agg_full(hh_, sd_, ete_, H_):
        pa, sa = _sc_aggregate(hh_, sd_, src2d, dst2d, et2d, ew2d, ete_,
                               zeros_pad, zeros8, H_, 0, half)
        pb, sb_ = _sc_aggregate(hh_, sd_, src2d, dst2d, et2d, ew2d, ete_,
                                zeros_pad, zeros8, H_, half, half)
        return (jnp.concatenate([pa, pb], axis=1),
                jnp.concatenate([sa.reshape(NW, half, 8),
                                 sb_.reshape(NW, half, 8)], axis=1))


    h0, sd0 = _tc_pre(x, W0.T.astype(F32), A0, bn)
    praw0, psum0 = agg_full(h0, pad_sd(sd0), ete0_1, 8)
    act0, h1, sd1 = _tc_mid(praw0, psum0, x, g0r, b0r, R8,
                            W1.T.astype(F32), A1, bn)
    praw1, psum1 = agg_full(h1, pad_sd(sd1), ete1_1, 8)
    act1, h2, sd2 = _tc_mid(praw1, psum1, act0, g1r, b1r, R8,
                            W2.T.astype(F32), A2, bn)
    praw2, psum2 = agg_full(h2, pad_sd(sd2), ete2_1, 1)
    hz0, hz1, surv, cif0, cif1 = _tc_head(
        praw2, psum2, act1, g2r, b2r, R1,
        sW1.T.astype(F32), sb1.reshape(1, 128),
        sW2.T.astype(F32), sb2.reshape(1, 128),
        rW1_0.T.astype(F32), rb1_0.reshape(1, 64),
        rW2_0.T.astype(F32), rb2_0.reshape(1, T),
        rW1_1.T.astype(F32), rb1_1.reshape(1, 64),
        rW2_1.T.astype(F32), rb2_1.reshape(1, T),
        baseline[0].reshape(1, T), baseline[1].reshape(1, T),
        U, Z, e0row, bn)

    hazards = jnp.stack([hz0, hz1], axis=1)
    cif = jnp.stack([cif0, cif1], axis=1)
    return hazards, surv, cif
